# baseline (speedup 1.0000x reference)
import functools

import jax
import jax.numpy as jnp
from jax import lax
from jax.experimental import pallas as pl
from jax.experimental.pallas import tpu as pltpu
from jax.experimental.pallas import tpu_sc as plsc

VOCAB = 151936
DIM = 1536
B = 4 * 2048
NUM_WORKERS = 32
B_PER_W = B // NUM_WORKERS
CHUNK = 16
NCHUNK = B_PER_W // CHUNK
NBUF = 4


def _gather_kernel(idx_hbm, table_hbm, out_hbm, idx_v, bufs_v, gsem, osem):
    wid = lax.axis_index("s") * 2 + lax.axis_index("c")
    base = wid * B_PER_W
    pltpu.sync_copy(idx_hbm.at[pl.ds(base, B_PER_W)], idx_v)

    for b in range(NBUF):
        pltpu.async_copy(
            table_hbm.at[idx_v.at[pl.ds(b * CHUNK, CHUNK)]],
            bufs_v.at[b], gsem.at[b])

    @pl.loop(0, NCHUNK)
    def _steady(i):
        b = lax.rem(i, NBUF)
        buf = bufs_v.at[b]
        pltpu.make_async_copy(
            table_hbm.at[idx_v.at[pl.ds(i * CHUNK, CHUNK)]],
            buf, gsem.at[b]).wait()
        out_slice = out_hbm.at[pl.ds(base + i * CHUNK, CHUNK)]
        pltpu.async_copy(buf, out_slice, osem.at[b])
        pltpu.make_async_copy(buf, out_slice, osem.at[b]).wait()

        @pl.when(i + NBUF < NCHUNK)
        def _():
            pltpu.async_copy(
                table_hbm.at[idx_v.at[pl.ds((i + NBUF) * CHUNK, CHUNK)]],
                buf, gsem.at[b])


@jax.jit
def kernel(input_ids, table):
    idx = input_ids.reshape(-1).astype(jnp.int32)
    mesh = plsc.VectorSubcoreMesh(core_axis_name="c", subcore_axis_name="s")
    run = functools.partial(
        pl.kernel,
        mesh=mesh,
        out_type=jax.ShapeDtypeStruct((B, DIM), jnp.float32),
        scratch_types=[
            pltpu.VMEM((B_PER_W,), jnp.int32),
            pltpu.VMEM((NBUF, CHUNK, DIM), jnp.float32),
            pltpu.SemaphoreType.DMA((NBUF,)),
            pltpu.SemaphoreType.DMA((NBUF,)),
        ],
    )(_gather_kernel)
    out = run(idx, table)
    return out.reshape(input_ids.shape + (DIM,))
